# baseline (speedup 1.0000x reference)
import functools

import jax
import jax.numpy as jnp
from jax import lax
from jax.experimental import pallas as pl
from jax.experimental.pallas import tpu as pltpu
from jax.experimental.pallas import tpu_sc as plsc

N = 10000
E = 320000
D = 128
G = 512

NC = 2
NS = 16
NW = NC * NS

CH = 128
ECH = 2560
WCH = ECH // NW
NPAD = 10240
GPAD = 768
BCH = NPAD // CH
ROWS_PER_SUB = NPAD // NS
PROWS = 640
PROWS_PER_SUB = PROWS // NS

@functools.lru_cache(maxsize=None)
def _mesh():
  return plsc.VectorSubcoreMesh(
      core_axis_name="c", subcore_axis_name="s", num_cores=NC,
      num_subcores=NS)

_f32 = jnp.float32


def _zero_rows(rows_ref, nwords):
  zero16 = jnp.zeros((16,), _f32)

  def body(i, carry):
    r = i // 8
    col = (i % 8) * 16
    rows_ref[r, pl.ds(col, 16)] = zero16
    return carry

  lax.fori_loop(0, nwords // 16, body, 0)


ECH_PER_SUB = ECH // NS
BCH_PER_SUB = BCH // NS


@functools.lru_cache(maxsize=None)
def _build_sc_hist():
  return functools.partial(
      pl.kernel,
      out_type=(
          jax.ShapeDtypeStruct((NPAD,), _f32),
          jax.ShapeDtypeStruct((GPAD,), _f32),
      ),
      mesh=_mesh(),
      scratch_types=[
          pltpu.VMEM((ECH_PER_SUB, CH), jnp.int32),
          pltpu.VMEM((BCH_PER_SUB, CH), jnp.int32),
          pltpu.VMEM((GPAD,), _f32),
          pltpu.VMEM((CH,), _f32),
          pltpu.VMEM_SHARED((NPAD,), _f32),
      ],
  )(_sc_hist_body)


def _sc_hist(dstp, bvp):
  return _build_sc_hist()(dstp, bvp)


def _sc_hist_body(dstp, bvp, dinv_out, icnt_out, didx, bidx, buf, ones,
                  acc):
  c = lax.axis_index("c")
  s = lax.axis_index("s")

  one16 = jnp.ones((16,), _f32)
  zero16 = jnp.zeros((16,), _f32)
  for k in range(CH // 16):
    ones[pl.ds(k * 16, 16)] = one16

  def zb(i, carry):
    buf[pl.ds(i * 16, 16)] = zero16
    return carry

  lax.fori_loop(0, GPAD // 16, zb, 0)

  if True:
    @pl.when(c == 0)
    def _():
      pltpu.sync_copy(buf.at[pl.ds(0, ROWS_PER_SUB)],
                      acc.at[pl.ds(s * ROWS_PER_SUB, ROWS_PER_SUB)])
      pltpu.sync_copy(dstp.at[pl.ds(s * ECH_PER_SUB, ECH_PER_SUB)], didx)

    @pl.when((c == 1) & (s == 0))
    def _():
      pltpu.sync_copy(buf.at[pl.ds(0, GPAD)], acc.at[pl.ds(0, GPAD)])

    @pl.when(c == 1)
    def _():
      for j in range(BCH_PER_SUB):
        pltpu.sync_copy(bvp.at[pl.ds((s * BCH_PER_SUB + j) * CH, CH)],
                        bidx.at[j])

    plsc.subcore_barrier()

    @pl.when(c == 0)
    def _():
      def deg_body(j, carry):
        pltpu.sync_copy(ones, acc.at[didx.at[j]], add=True)
        return carry

      lax.fori_loop(0, ECH_PER_SUB, deg_body, 0)

    @pl.when(c == 1)
    def _():
      for j in range(BCH_PER_SUB):
        pltpu.sync_copy(ones, acc.at[bidx.at[j]], add=True)

    plsc.subcore_barrier()

    @pl.when(c == 0)
    def _():
      pltpu.sync_copy(acc.at[pl.ds(s * ROWS_PER_SUB, ROWS_PER_SUB)],
                      buf.at[pl.ds(0, ROWS_PER_SUB)])

      def rs_body(i, carry):
        v = buf[pl.ds(i * 16, 16)] + 1.0
        bits = lax.bitcast_convert_type(v, jnp.int32)
        y0 = lax.bitcast_convert_type(
            jnp.int32(0x5F3759DF) - lax.shift_right_logical(bits, 1), _f32)
        for _ in range(3):
          y0 = y0 * (1.5 - 0.5 * v * y0 * y0)
        buf[pl.ds(i * 16, 16)] = y0
        return carry

      lax.fori_loop(0, ROWS_PER_SUB // 16, rs_body, 0)
      pltpu.sync_copy(buf.at[pl.ds(0, ROWS_PER_SUB)],
                      dinv_out.at[pl.ds(s * ROWS_PER_SUB, ROWS_PER_SUB)])

    @pl.when(c == 1)
    def _():
      span = GPAD // NS
      pltpu.sync_copy(acc.at[pl.ds(s * span, span)], buf.at[pl.ds(0, span)])
      for i in range(span // 16):
        cv = buf[pl.ds(i * 16, 16)]
        buf[pl.ds(i * 16, 16)] = 1.0 / jnp.maximum(cv, 1.0)
      pltpu.sync_copy(buf.at[pl.ds(0, span)],
                      icnt_out.at[pl.ds(s * span, span)])


@functools.lru_cache(maxsize=None)
def _build_sc_agg():
  return functools.partial(
      pl.kernel,
      out_type=jax.ShapeDtypeStruct((NC, NPAD, D), _f32),
      mesh=_mesh(),
      scratch_types=[
          pltpu.VMEM((WCH // 2, CH), jnp.int32),
          pltpu.VMEM((WCH // 2, CH), jnp.int32),
          pltpu.VMEM((CH, D), _f32),
          pltpu.VMEM((CH, D), _f32),
          pltpu.SemaphoreType.DMA,
          pltpu.SemaphoreType.DMA,
          pltpu.SemaphoreType.DMA,
          pltpu.SemaphoreType.DMA,
          pltpu.VMEM_SHARED((NPAD, D), _f32),
      ],
  )(_sc_agg_body)


def _sc_agg(y, srcp, dstp):
  return _build_sc_agg()(y, srcp, dstp)


def _sc_agg_body(y, srcp, dstp, s_out, sidx, didx, rows_a, rows_b, sem_a,
                 sem_b, sem_sa, sem_sb, acc):
  c = lax.axis_index("c")
  s = lax.axis_index("s")
  w = s * NC + c

  _zero_rows(rows_a, CH * D)

  for k in range(ROWS_PER_SUB // CH):
    pltpu.sync_copy(rows_a, acc.at[pl.ds(s * ROWS_PER_SUB + k * CH, CH)])
  plsc.subcore_barrier()

  HALF = WCH // 2

  for h in range(2):
    pltpu.sync_copy(srcp.at[pl.ds(w * WCH + h * HALF, HALF)], sidx)
    pltpu.sync_copy(dstp.at[pl.ds(w * WCH + h * HALF, HALF)], didx)

    pltpu.async_copy(y.at[sidx.at[0]], rows_a, sem_a)
    pltpu.async_copy(y.at[sidx.at[1]], rows_b, sem_b)

    def pair(jp, carry):
      j0 = jp * 2
      pltpu.make_async_copy(y.at[sidx.at[j0]], rows_a, sem_a).wait()
      pltpu.async_copy(rows_a, acc.at[didx.at[j0]], sem_sa, add=True)
      pltpu.make_async_copy(y.at[sidx.at[j0 + 1]], rows_b, sem_b).wait()
      pltpu.async_copy(rows_b, acc.at[didx.at[j0 + 1]], sem_sb, add=True)

      pltpu.make_async_copy(rows_a, acc.at[didx.at[j0]], sem_sa).wait()

      @pl.when(jp < HALF // 2 - 1)
      def _():
        pltpu.async_copy(y.at[sidx.at[j0 + 2]], rows_a, sem_a)

      pltpu.make_async_copy(rows_b, acc.at[didx.at[j0 + 1]], sem_sb).wait()

      @pl.when(jp < HALF // 2 - 1)
      def _():
        pltpu.async_copy(y.at[sidx.at[j0 + 3]], rows_b, sem_b)

      return carry

    lax.fori_loop(0, HALF // 2, pair, 0)

  plsc.subcore_barrier()

  base = s * ROWS_PER_SUB
  for k in range(ROWS_PER_SUB // CH):
    pltpu.sync_copy(acc.at[pl.ds(base + k * CH, CH)], rows_a)
    pltpu.sync_copy(rows_a, s_out.at[c, pl.ds(base + k * CH, CH)])


@functools.lru_cache(maxsize=None)
def _build_sc_pool():
  return functools.partial(
      pl.kernel,
      out_type=jax.ShapeDtypeStruct((NC, PROWS, D), _f32),
      mesh=_mesh(),
      scratch_types=[
          pltpu.VMEM((3, CH), jnp.int32),
          pltpu.VMEM((CH, D), _f32),
          pltpu.VMEM_SHARED((PROWS, D), _f32),
      ],
  )(_sc_pool_body)


def _sc_pool(hp, bvp):
  return _build_sc_pool()(hp, bvp)


def _sc_pool_body(hp, bvp, p_out, bidx, rows, acc):
  c = lax.axis_index("c")
  s = lax.axis_index("s")
  w = s * NC + c

  pltpu.sync_copy(bvp.at[pl.ds(w * CH, CH)], bidx.at[0])
  pltpu.sync_copy(bvp.at[pl.ds((w + 32) * CH, CH)], bidx.at[1])

  @pl.when(w < BCH - 64)
  def _():
    pltpu.sync_copy(bvp.at[pl.ds((w + 64) * CH, CH)], bidx.at[2])

  _zero_rows(rows, CH * D)

  if True:
    pltpu.sync_copy(rows.at[pl.ds(0, PROWS_PER_SUB)],
                    acc.at[pl.ds(s * PROWS_PER_SUB, PROWS_PER_SUB)])
    plsc.subcore_barrier()

    def chunk(k):
      cid = w + 32 * k
      pltpu.sync_copy(hp.at[pl.ds(cid * CH, CH)], rows)
      pltpu.sync_copy(rows, acc.at[bidx.at[k]], add=True)

    chunk(0)
    chunk(1)

    @pl.when(w < BCH - 64)
    def _():
      chunk(2)

    plsc.subcore_barrier()

    pltpu.sync_copy(acc.at[pl.ds(s * PROWS_PER_SUB, PROWS_PER_SUB)],
                    rows.at[pl.ds(0, PROWS_PER_SUB)])
    pltpu.sync_copy(rows.at[pl.ds(0, PROWS_PER_SUB)],
                    p_out.at[c, pl.ds(s * PROWS_PER_SUB, PROWS_PER_SUB)])


_BLK = 400


def _tc_prep_body(dinv_ref, x_ref, w1_ref, y_ref):
  y_ref[...] = jnp.dot(dinv_ref[...] * x_ref[...], w1_ref[...],
                       preferred_element_type=_f32)


def _tc_prep(dinv, x, w1):
  return pl.pallas_call(
      _tc_prep_body,
      grid=(N // _BLK,),
      in_specs=[
          pl.BlockSpec((_BLK, 1), lambda g: (g, 0)),
          pl.BlockSpec((_BLK, D), lambda g: (g, 0)),
          pl.BlockSpec((D, D), lambda g: (0, 0)),
      ],
      out_specs=pl.BlockSpec((_BLK, D), lambda g: (g, 0)),
      out_shape=jax.ShapeDtypeStruct((N, D), _f32),
  )(dinv, x, w1)


def _tc_mid_body(s_ref, y_ref, dinv_ref, b_ref, w_ref, out_ref):
  dv = dinv_ref[...]
  h = jnp.maximum(dv * (s_ref[0] + s_ref[1] + y_ref[...]) + b_ref[...], 0.0)
  out_ref[...] = jnp.dot(dv * h, w_ref[...], preferred_element_type=_f32)


def _tc_mid(s2, y_prev, dinv, b_prev, w_next):
  return pl.pallas_call(
      _tc_mid_body,
      grid=(N // _BLK,),
      in_specs=[
          pl.BlockSpec((NC, _BLK, D), lambda g: (0, g, 0)),
          pl.BlockSpec((_BLK, D), lambda g: (g, 0)),
          pl.BlockSpec((_BLK, 1), lambda g: (g, 0)),
          pl.BlockSpec((1, D), lambda g: (0, 0)),
          pl.BlockSpec((D, D), lambda g: (0, 0)),
      ],
      out_specs=pl.BlockSpec((_BLK, D), lambda g: (g, 0)),
      out_shape=jax.ShapeDtypeStruct((N, D), _f32),
  )(s2, y_prev, dinv, b_prev, w_next)


def _tc_fin_body(s_ref, y_ref, dinv_ref, b_ref, out_ref):
  dv = dinv_ref[...]
  out_ref[...] = jnp.maximum(
      dv * (s_ref[0] + s_ref[1] + y_ref[...]) + b_ref[...], 0.0)


def _tc_fin(s2, y3, dinv, b3):
  return pl.pallas_call(
      _tc_fin_body,
      grid=(N // _BLK,),
      in_specs=[
          pl.BlockSpec((NC, _BLK, D), lambda g: (0, g, 0)),
          pl.BlockSpec((_BLK, D), lambda g: (g, 0)),
          pl.BlockSpec((_BLK, 1), lambda g: (g, 0)),
          pl.BlockSpec((1, D), lambda g: (0, 0)),
      ],
      out_specs=pl.BlockSpec((_BLK, D), lambda g: (g, 0)),
      out_shape=jax.ShapeDtypeStruct((N, D), _f32),
  )(s2, y3, dinv, b3)


def _tc_head_body(p_ref, icnt_ref, wf1_ref, bf1_ref, wf2_ref, bf2_ref,
                  out_ref):
  pooled = (p_ref[0, :G, :] + p_ref[1, :G, :]) * icnt_ref[...]
  z = jnp.maximum(
      jnp.dot(pooled, wf1_ref[...], preferred_element_type=_f32)
      + bf1_ref[...], 0.0)
  out_ref[...] = (jnp.dot(z, wf2_ref[...], preferred_element_type=_f32)
                  + bf2_ref[...])


def _tc_head(p2, icnt, wf1, bf1, wf2, bf2):
  return pl.pallas_call(
      _tc_head_body,
      out_shape=jax.ShapeDtypeStruct((G, wf2.shape[1]), _f32),
  )(p2, icnt, wf1, bf1, wf2, bf2)


def kernel(x, edge_index, batch_vec, W1, b1, W2, b2, W3, b3,
           Wf1, bf1, Wf2, bf2):
  pad_e = ECH * CH - E
  src_p = jnp.concatenate(
      [edge_index[0],
       (jnp.arange(pad_e, dtype=jnp.int32) * 67) % N]).reshape(ECH, CH)
  dst_p = jnp.concatenate(
      [edge_index[1],
       N + jnp.arange(pad_e, dtype=jnp.int32) % 16]).reshape(ECH, CH)
  pad_b = NPAD - N
  bv_p = jnp.concatenate(
      [batch_vec, G + jnp.arange(pad_b, dtype=jnp.int32) % 16])

  dinv_full, icnt_full = _sc_hist(dst_p, bv_p)
  dinv = dinv_full[:N].reshape(N, 1)
  icnt = icnt_full[:G].reshape(G, 1)
  y1 = _tc_prep(dinv, x, W1)

  s1 = _sc_agg(y1, src_p, dst_p)
  y2 = _tc_mid(s1[:, :N], y1, dinv, b1.reshape(1, D), W2)
  s2 = _sc_agg(y2, src_p, dst_p)
  y3 = _tc_mid(s2[:, :N], y2, dinv, b2.reshape(1, D), W3)
  s3 = _sc_agg(y3, src_p, dst_p)
  h3 = _tc_fin(s3[:, :N], y3, dinv, b3.reshape(1, D))

  hp = jnp.concatenate([h3, jnp.zeros((pad_b, D), _f32)], axis=0)
  p2 = _sc_pool(hp, bv_p)

  nf2 = Wf2.shape[1]
  nf2p = 32
  wf2p = jnp.concatenate(
      [Wf2, jnp.zeros((Wf2.shape[0], nf2p - nf2), _f32)], axis=1)
  bf2p = jnp.concatenate([bf2, jnp.zeros((nf2p - nf2,), _f32)])

  out = _tc_head(p2, icnt, Wf1, bf1.reshape(1, -1), wf2p,
                 bf2p.reshape(1, -1))
  return out[:, :nf2]


# speedup vs baseline: 1.2748x; 1.2748x over previous
import functools

import jax
import jax.numpy as jnp
from jax import lax
from jax.experimental import pallas as pl
from jax.experimental.pallas import tpu as pltpu
from jax.experimental.pallas import tpu_sc as plsc

N = 10000
E = 320000
D = 128
G = 512

NC = 2
NS = 16
NW = NC * NS

CH = 128
ECH = 2560
WCH = ECH // NW
NPAD = 10240
GPAD = 768
BCH = NPAD // CH
ROWS_PER_SUB = NPAD // NS
PROWS = 640
PROWS_PER_SUB = PROWS // NS

@functools.lru_cache(maxsize=None)
def _mesh():
  return plsc.VectorSubcoreMesh(
      core_axis_name="c", subcore_axis_name="s", num_cores=NC,
      num_subcores=NS)

_f32 = jnp.float32


def _zero_rows(rows_ref, nwords):
  zero16 = jnp.zeros((16,), _f32)

  def body(i, carry):
    r = i // 8
    col = (i % 8) * 16
    rows_ref[r, pl.ds(col, 16)] = zero16
    return carry

  lax.fori_loop(0, nwords // 16, body, 0)


ECH_PER_SUB = ECH // NS
BCH_PER_SUB = BCH // NS


@functools.lru_cache(maxsize=None)
def _build_sc_hist():
  return functools.partial(
      pl.kernel,
      out_type=(
          jax.ShapeDtypeStruct((NPAD,), _f32),
          jax.ShapeDtypeStruct((GPAD,), _f32),
      ),
      mesh=_mesh(),
      scratch_types=[
          pltpu.VMEM((ECH_PER_SUB, CH), jnp.int32),
          pltpu.VMEM((BCH_PER_SUB, CH), jnp.int32),
          pltpu.VMEM((GPAD,), _f32),
          pltpu.VMEM((CH,), _f32),
          pltpu.VMEM_SHARED((NPAD,), _f32),
      ],
  )(_sc_hist_body)


def _sc_hist(dstp, bvp):
  return _build_sc_hist()(dstp, bvp)


def _sc_hist_body(dstp, bvp, dinv_out, icnt_out, didx, bidx, buf, ones,
                  acc):
  c = lax.axis_index("c")
  s = lax.axis_index("s")

  one16 = jnp.ones((16,), _f32)
  zero16 = jnp.zeros((16,), _f32)
  for k in range(CH // 16):
    ones[pl.ds(k * 16, 16)] = one16

  def zb(i, carry):
    buf[pl.ds(i * 16, 16)] = zero16
    return carry

  lax.fori_loop(0, GPAD // 16, zb, 0)

  if True:
    @pl.when(c == 0)
    def _():
      pltpu.sync_copy(buf.at[pl.ds(0, ROWS_PER_SUB)],
                      acc.at[pl.ds(s * ROWS_PER_SUB, ROWS_PER_SUB)])
      pltpu.sync_copy(dstp.at[pl.ds(s * ECH_PER_SUB, ECH_PER_SUB)], didx)

    @pl.when((c == 1) & (s == 0))
    def _():
      pltpu.sync_copy(buf.at[pl.ds(0, GPAD)], acc.at[pl.ds(0, GPAD)])

    @pl.when(c == 1)
    def _():
      for j in range(BCH_PER_SUB):
        pltpu.sync_copy(bvp.at[pl.ds((s * BCH_PER_SUB + j) * CH, CH)],
                        bidx.at[j])

    plsc.subcore_barrier()

    @pl.when(c == 0)
    def _():
      def deg_body(j, carry):
        pltpu.sync_copy(ones, acc.at[didx.at[j]], add=True)
        return carry

      lax.fori_loop(0, ECH_PER_SUB, deg_body, 0)

    @pl.when(c == 1)
    def _():
      for j in range(BCH_PER_SUB):
        pltpu.sync_copy(ones, acc.at[bidx.at[j]], add=True)

    plsc.subcore_barrier()

    @pl.when(c == 0)
    def _():
      pltpu.sync_copy(acc.at[pl.ds(s * ROWS_PER_SUB, ROWS_PER_SUB)],
                      buf.at[pl.ds(0, ROWS_PER_SUB)])

      def rs_body(i, carry):
        v = buf[pl.ds(i * 16, 16)] + 1.0
        bits = lax.bitcast_convert_type(v, jnp.int32)
        y0 = lax.bitcast_convert_type(
            jnp.int32(0x5F3759DF) - lax.shift_right_logical(bits, 1), _f32)
        for _ in range(3):
          y0 = y0 * (1.5 - 0.5 * v * y0 * y0)
        buf[pl.ds(i * 16, 16)] = y0
        return carry

      lax.fori_loop(0, ROWS_PER_SUB // 16, rs_body, 0)
      pltpu.sync_copy(buf.at[pl.ds(0, ROWS_PER_SUB)],
                      dinv_out.at[pl.ds(s * ROWS_PER_SUB, ROWS_PER_SUB)])

    @pl.when(c == 1)
    def _():
      span = GPAD // NS
      pltpu.sync_copy(acc.at[pl.ds(s * span, span)], buf.at[pl.ds(0, span)])
      for i in range(span // 16):
        cv = buf[pl.ds(i * 16, 16)]
        buf[pl.ds(i * 16, 16)] = 1.0 / jnp.maximum(cv, 1.0)
      pltpu.sync_copy(buf.at[pl.ds(0, span)],
                      icnt_out.at[pl.ds(s * span, span)])


@functools.lru_cache(maxsize=None)
def _build_sc_agg():
  return functools.partial(
      pl.kernel,
      out_type=jax.ShapeDtypeStruct((NC, NPAD, D), _f32),
      mesh=_mesh(),
      scratch_types=[
          pltpu.VMEM((WCH // 2, CH), jnp.int32),
          pltpu.VMEM((WCH // 2, CH), jnp.int32),
          pltpu.VMEM((CH, D), _f32),
          pltpu.VMEM((CH, D), _f32),
          pltpu.SemaphoreType.DMA,
          pltpu.SemaphoreType.DMA,
          pltpu.VMEM_SHARED((NPAD, D), _f32),
      ],
  )(_sc_agg_body)


def _sc_agg(y, srcp, dstp):
  return _build_sc_agg()(y, srcp, dstp)


def _sc_agg_body(y, srcp, dstp, s_out, sidx, didx, rows_a, rows_b, sem_a,
                 sem_b, acc):
  c = lax.axis_index("c")
  s = lax.axis_index("s")
  w = s * NC + c

  _zero_rows(rows_a, CH * D)

  for k in range(ROWS_PER_SUB // CH):
    pltpu.sync_copy(rows_a, acc.at[pl.ds(s * ROWS_PER_SUB + k * CH, CH)])
  plsc.subcore_barrier()

  HALF = WCH // 2

  for h in range(2):
    pltpu.sync_copy(srcp.at[pl.ds(w * WCH + h * HALF, HALF)], sidx)
    pltpu.sync_copy(dstp.at[pl.ds(w * WCH + h * HALF, HALF)], didx)

    pltpu.async_copy(y.at[sidx.at[0]], rows_a, sem_a)
    pltpu.async_copy(y.at[sidx.at[1]], rows_b, sem_b)

    def pair(jp, carry):
      j0 = jp * 2
      pltpu.make_async_copy(y.at[sidx.at[j0]], rows_a, sem_a).wait()
      pltpu.sync_copy(rows_a, acc.at[didx.at[j0]], add=True)

      @pl.when(jp < HALF // 2 - 1)
      def _():
        pltpu.async_copy(y.at[sidx.at[j0 + 2]], rows_a, sem_a)

      pltpu.make_async_copy(y.at[sidx.at[j0 + 1]], rows_b, sem_b).wait()
      pltpu.sync_copy(rows_b, acc.at[didx.at[j0 + 1]], add=True)

      @pl.when(jp < HALF // 2 - 1)
      def _():
        pltpu.async_copy(y.at[sidx.at[j0 + 3]], rows_b, sem_b)

      return carry

    lax.fori_loop(0, HALF // 2, pair, 0)

  plsc.subcore_barrier()

  base = s * ROWS_PER_SUB
  for k in range(ROWS_PER_SUB // CH):
    pltpu.sync_copy(acc.at[pl.ds(base + k * CH, CH)], rows_a)
    pltpu.sync_copy(rows_a, s_out.at[c, pl.ds(base + k * CH, CH)])


@functools.lru_cache(maxsize=None)
def _build_sc_pool():
  return functools.partial(
      pl.kernel,
      out_type=jax.ShapeDtypeStruct((NC, PROWS, D), _f32),
      mesh=_mesh(),
      scratch_types=[
          pltpu.VMEM((3, CH), jnp.int32),
          pltpu.VMEM((CH, D), _f32),
          pltpu.VMEM_SHARED((PROWS, D), _f32),
      ],
  )(_sc_pool_body)


def _sc_pool(hp, bvp):
  return _build_sc_pool()(hp, bvp)


def _sc_pool_body(hp, bvp, p_out, bidx, rows, acc):
  c = lax.axis_index("c")
  s = lax.axis_index("s")
  w = s * NC + c

  pltpu.sync_copy(bvp.at[pl.ds(w * CH, CH)], bidx.at[0])
  pltpu.sync_copy(bvp.at[pl.ds((w + 32) * CH, CH)], bidx.at[1])

  @pl.when(w < BCH - 64)
  def _():
    pltpu.sync_copy(bvp.at[pl.ds((w + 64) * CH, CH)], bidx.at[2])

  _zero_rows(rows, CH * D)

  if True:
    pltpu.sync_copy(rows.at[pl.ds(0, PROWS_PER_SUB)],
                    acc.at[pl.ds(s * PROWS_PER_SUB, PROWS_PER_SUB)])
    plsc.subcore_barrier()

    def chunk(k):
      cid = w + 32 * k
      pltpu.sync_copy(hp.at[pl.ds(cid * CH, CH)], rows)
      pltpu.sync_copy(rows, acc.at[bidx.at[k]], add=True)

    chunk(0)
    chunk(1)

    @pl.when(w < BCH - 64)
    def _():
      chunk(2)

    plsc.subcore_barrier()

    pltpu.sync_copy(acc.at[pl.ds(s * PROWS_PER_SUB, PROWS_PER_SUB)],
                    rows.at[pl.ds(0, PROWS_PER_SUB)])
    pltpu.sync_copy(rows.at[pl.ds(0, PROWS_PER_SUB)],
                    p_out.at[c, pl.ds(s * PROWS_PER_SUB, PROWS_PER_SUB)])


_BLK = 400


def _tc_prep_body(dinv_ref, x_ref, w1_ref, y_ref):
  y_ref[...] = jnp.dot(dinv_ref[...] * x_ref[...], w1_ref[...],
                       preferred_element_type=_f32)


def _tc_prep(dinv, x, w1):
  return pl.pallas_call(
      _tc_prep_body,
      grid=(N // _BLK,),
      in_specs=[
          pl.BlockSpec((_BLK, 1), lambda g: (g, 0)),
          pl.BlockSpec((_BLK, D), lambda g: (g, 0)),
          pl.BlockSpec((D, D), lambda g: (0, 0)),
      ],
      out_specs=pl.BlockSpec((_BLK, D), lambda g: (g, 0)),
      out_shape=jax.ShapeDtypeStruct((N, D), _f32),
  )(dinv, x, w1)


def _tc_mid_body(s_ref, y_ref, dinv_ref, b_ref, w_ref, out_ref):
  dv = dinv_ref[...]
  h = jnp.maximum(dv * (s_ref[0] + s_ref[1] + y_ref[...]) + b_ref[...], 0.0)
  out_ref[...] = jnp.dot(dv * h, w_ref[...], preferred_element_type=_f32)


def _tc_mid(s2, y_prev, dinv, b_prev, w_next):
  return pl.pallas_call(
      _tc_mid_body,
      grid=(N // _BLK,),
      in_specs=[
          pl.BlockSpec((NC, _BLK, D), lambda g: (0, g, 0)),
          pl.BlockSpec((_BLK, D), lambda g: (g, 0)),
          pl.BlockSpec((_BLK, 1), lambda g: (g, 0)),
          pl.BlockSpec((1, D), lambda g: (0, 0)),
          pl.BlockSpec((D, D), lambda g: (0, 0)),
      ],
      out_specs=pl.BlockSpec((_BLK, D), lambda g: (g, 0)),
      out_shape=jax.ShapeDtypeStruct((N, D), _f32),
  )(s2, y_prev, dinv, b_prev, w_next)


def _tc_fin_body(s_ref, y_ref, dinv_ref, b_ref, out_ref):
  dv = dinv_ref[...]
  out_ref[...] = jnp.maximum(
      dv * (s_ref[0] + s_ref[1] + y_ref[...]) + b_ref[...], 0.0)


def _tc_fin(s2, y3, dinv, b3):
  return pl.pallas_call(
      _tc_fin_body,
      grid=(N // _BLK,),
      in_specs=[
          pl.BlockSpec((NC, _BLK, D), lambda g: (0, g, 0)),
          pl.BlockSpec((_BLK, D), lambda g: (g, 0)),
          pl.BlockSpec((_BLK, 1), lambda g: (g, 0)),
          pl.BlockSpec((1, D), lambda g: (0, 0)),
      ],
      out_specs=pl.BlockSpec((_BLK, D), lambda g: (g, 0)),
      out_shape=jax.ShapeDtypeStruct((NPAD, D), _f32),
  )(s2, y3, dinv, b3)


def _tc_head_body(p_ref, icnt_ref, wf1_ref, bf1_ref, wf2_ref, bf2_ref,
                  out_ref):
  pooled = (p_ref[0, :G, :] + p_ref[1, :G, :]) * icnt_ref[...]
  z = jnp.maximum(
      jnp.dot(pooled, wf1_ref[...], preferred_element_type=_f32)
      + bf1_ref[...], 0.0)
  out_ref[...] = (jnp.dot(z, wf2_ref[...], preferred_element_type=_f32)
                  + bf2_ref[...])


def _tc_head(p2, icnt, wf1, bf1, wf2, bf2):
  return pl.pallas_call(
      _tc_head_body,
      out_shape=jax.ShapeDtypeStruct((G, wf2.shape[1]), _f32),
  )(p2, icnt, wf1, bf1, wf2, bf2)


def kernel(x, edge_index, batch_vec, W1, b1, W2, b2, W3, b3,
           Wf1, bf1, Wf2, bf2):
  pad_e = ECH * CH - E
  src_p = jnp.concatenate(
      [edge_index[0],
       (jnp.arange(pad_e, dtype=jnp.int32) * 67) % N]).reshape(ECH, CH)
  dst_p = jnp.concatenate(
      [edge_index[1],
       N + jnp.arange(pad_e, dtype=jnp.int32) % 16]).reshape(ECH, CH)
  pad_b = NPAD - N
  bv_p = jnp.concatenate(
      [batch_vec, G + jnp.arange(pad_b, dtype=jnp.int32) % 16])

  dinv_full, icnt_full = _sc_hist(dst_p, bv_p)
  dinv = dinv_full[:N].reshape(N, 1)
  icnt = icnt_full[:G].reshape(G, 1)
  y1 = _tc_prep(dinv, x, W1)

  s1 = _sc_agg(y1, src_p, dst_p)
  y2 = _tc_mid(s1, y1, dinv, b1.reshape(1, D), W2)
  s2 = _sc_agg(y2, src_p, dst_p)
  y3 = _tc_mid(s2, y2, dinv, b2.reshape(1, D), W3)
  s3 = _sc_agg(y3, src_p, dst_p)
  hp = _tc_fin(s3, y3, dinv, b3.reshape(1, D))
  p2 = _sc_pool(hp, bv_p)

  nf2 = Wf2.shape[1]
  nf2p = 32
  wf2p = jnp.concatenate(
      [Wf2, jnp.zeros((Wf2.shape[0], nf2p - nf2), _f32)], axis=1)
  bf2p = jnp.concatenate([bf2, jnp.zeros((nf2p - nf2,), _f32)])

  out = _tc_head(p2, icnt, Wf1, bf1.reshape(1, -1), wf2p,
                 bf2p.reshape(1, -1))
  return out[:, :nf2]
